# baseline (speedup 1.0000x reference)
import functools

import jax
import jax.numpy as jnp
from jax import lax
from jax.experimental import pallas as pl
from jax.experimental.pallas import tpu as pltpu
from jax.experimental.pallas import tpu_sc as plsc

F32 = jnp.float32
N0_, N1_, N2_ = 10000, 4000, 1024
E1_, E2_ = 320000, 128000
F_IN_, H_, C_ = 128, 64, 10

NC, NS = 2, 16
NW = NC * NS
RW = 80


def _l1_dense_body(x_ref, w_ref, avs_ref, avd_ref, hs_ref, as_ref, ad_ref):
    hs = jnp.dot(x_ref[...], w_ref[...], preferred_element_type=F32)
    hs_ref[...] = hs
    as_ref[...] = jnp.dot(hs, avs_ref[...], preferred_element_type=F32)
    ad_ref[...] = jnp.dot(hs, avd_ref[...], preferred_element_type=F32)


def _mid_dense_body(acc_ref, b1_ref, w2_ref, avs_ref, avd_ref,
                    hs2_ref, as2_ref, ad2_ref):
    a0 = acc_ref[0, :N1_]
    a1 = acc_ref[1, :N1_]
    num = a0[:, :H_] + a1[:, :H_]
    den = a0[:, H_:H_ + 1] + a1[:, H_:H_ + 1]
    h = jnp.maximum(num / (den + 1e-16) + b1_ref[...], 0.0)
    hs2 = jnp.dot(h, w2_ref[...], preferred_element_type=F32)
    hs2_ref[...] = hs2
    as2_ref[...] = jnp.dot(hs2, avs_ref[...], preferred_element_type=F32)
    ad2_ref[...] = jnp.dot(hs2, avd_ref[...], preferred_element_type=F32)


def _final_dense_body(acc_ref, b2_ref, wl_ref, bl_ref, out_ref):
    a0 = acc_ref[0, :N2_]
    a1 = acc_ref[1, :N2_]
    num = a0[:, :H_] + a1[:, :H_]
    den = a0[:, H_:H_ + 1] + a1[:, H_:H_ + 1]
    h2 = num / (den + 1e-16) + b2_ref[...]
    logits = jnp.dot(h2, wl_ref[...], preferred_element_type=F32) + bl_ref[...]
    m = jnp.max(logits, axis=1, keepdims=True)
    lse = m + jnp.log(jnp.sum(jnp.exp(logits - m), axis=1, keepdims=True))
    out_ref[...] = logits - lse


def _make_edge_kernel(n_src, n_dst, n_edges, chunk):
    ew = n_edges // NW
    n_chunks = ew // chunk
    nbuf = 3
    assert ew % chunk == 0 and chunk % 8 == 0
    nq = n_chunks // nbuf
    rem = n_chunks % nbuf
    n_dst_pad = (n_dst + 127) // 128 * 128
    rpt = n_dst_pad // NS
    ZR = 32
    mesh = plsc.VectorSubcoreMesh(core_axis_name="c", subcore_axis_name="s")

    @functools.partial(
        pl.kernel,
        out_type=jax.ShapeDtypeStruct((NC, n_dst_pad, RW), F32),
        mesh=mesh,
        compiler_params=pltpu.CompilerParams(
            needs_layout_passes=False, use_tc_tiling_on_sc=False),
        scratch_types=[
            pltpu.VMEM((n_src,), F32),
            pltpu.VMEM((n_dst,), F32),
            [pltpu.VMEM((chunk,), jnp.int32)] * nbuf,
            [pltpu.VMEM((chunk,), jnp.int32)] * nbuf,
            [pltpu.VMEM((chunk, H_), F32)] * nbuf,
            [pltpu.VMEM((chunk, RW), F32)] * nbuf,
            pltpu.VMEM((ZR, RW), F32),
            pltpu.VMEM_SHARED((n_dst_pad, RW), F32),
            [pltpu.SemaphoreType.DMA] * nbuf,
            [pltpu.SemaphoreType.DMA] * nbuf,
        ],
    )
    def edge_kernel(hs_hbm, asrc_hbm, adst_hbm, src_hbm, dst_hbm, out_hbm,
                    asrc_v, adst_v, src_v, dst_v, rows_v, srows_v,
                    zero_v, acc_sh, gsem, ssem):
        c = lax.axis_index("c")
        s = lax.axis_index("s")
        wid = s * NC + c

        pltpu.sync_copy(asrc_hbm, asrc_v)
        pltpu.sync_copy(adst_hbm, adst_v)
        zv = jnp.zeros((16,), F32)

        def zrow(r, _):
            for col in range(RW // 16):
                zero_v[r, pl.ds(col * 16, 16)] = zv
            return 0
        lax.fori_loop(0, ZR, zrow, 0)

        def zpad(r, _):
            for b in range(nbuf):
                srows_v[b][r, pl.ds(H_, 16)] = zv
            return 0
        lax.fori_loop(0, chunk, zpad, 0)

        def zacc(i, _):
            pltpu.sync_copy(zero_v, acc_sh.at[pl.ds(s * rpt + i * ZR, ZR)])
            return 0
        lax.fori_loop(0, rpt // ZR, zacc, 0)
        plsc.subcore_barrier()

        lane = lax.iota(jnp.int32, 16)
        col64 = jnp.full((16,), H_, jnp.int32)
        e0 = wid * ew

        def load_idx(g, b):
            base = e0 + g * chunk
            pltpu.sync_copy(src_hbm.at[pl.ds(base, chunk)], src_v[b])
            pltpu.sync_copy(dst_hbm.at[pl.ds(base, chunk)], dst_v[b])

        def scale(b):
            def group(b16):
                s16 = src_v[b][pl.ds(b16, 16)]
                d16 = dst_v[b][pl.ds(b16, 16)]
                e = plsc.load_gather(asrc_v, [s16]) + plsc.load_gather(adst_v, [d16])
                e = jnp.where(e >= 0.0, e, e * 0.2)
                ex16 = jnp.exp(e)
                nc = H_ // 16
                for r0 in range(0, 16, 4):
                    vals = [rows_v[b][b16 + r0 + q, pl.ds(c * 16, 16)]
                            for q in range(4) for c in range(nc)]
                    prods = [vals[q * nc + c] * ex16[r0 + q]
                             for q in range(4) for c in range(nc)]
                    for q in range(4):
                        for c in range(nc):
                            srows_v[b][b16 + r0 + q, pl.ds(c * 16, 16)] = (
                                prods[q * nc + c])
                plsc.store_scatter(srows_v[b], [b16 + lane, col64], ex16)

            @plsc.parallel_loop(0, chunk // 16, unroll=4)
            def scale_body(j):
                group(j * 16)
            if chunk % 16:
                group(chunk - 16)

        def rotation(g0, nb):
            gathers = []
            for b in range(nb):
                load_idx(g0 + b, b)
                gathers.append(
                    pltpu.async_copy(hs_hbm.at[src_v[b]], rows_v[b], gsem[b]))
            scatters = []
            for b in range(nb):
                gathers[b].wait()
                scale(b)
                scatters.append(
                    pltpu.async_copy(srows_v[b], acc_sh.at[dst_v[b]], ssem[b],
                                     add=True))
            for b in range(nb):
                scatters[b].wait()

        def rot_body(t, _):
            rotation(nbuf * t, nbuf)
            return 0
        lax.fori_loop(0, nq, rot_body, 0)
        if rem:
            rotation(nq * nbuf, rem)

        plsc.subcore_barrier()

        def wb(i, _):
            pltpu.sync_copy(acc_sh.at[pl.ds(s * rpt + i * ZR, ZR)],
                            out_hbm.at[c, pl.ds(s * rpt + i * ZR, ZR)])
            return 0
        lax.fori_loop(0, rpt // ZR, wb, 0)

    return edge_kernel


_edge_l1 = _make_edge_kernel(N0_, N1_, E1_, 200)
_edge_l2 = _make_edge_kernel(N1_, N2_, E2_, 200)


def kernel(x, src1, dst1, src2, dst2, W1, a1s, a1d, b1, W2, a2s, a2d, b2, Wl, bl):
    hs1, as1, ad1 = pl.pallas_call(
        _l1_dense_body,
        out_shape=[
            jax.ShapeDtypeStruct((N0_, H_), F32),
            jax.ShapeDtypeStruct((N0_, 1), F32),
            jax.ShapeDtypeStruct((N0_, 1), F32),
        ],
    )(x, W1, a1s.reshape(H_, 1), a1d.reshape(H_, 1))

    acc1 = _edge_l1(hs1, as1.reshape(N0_), ad1.reshape(N0_)[:N1_], src1, dst1)

    hs2, as2, ad2 = pl.pallas_call(
        _mid_dense_body,
        out_shape=[
            jax.ShapeDtypeStruct((N1_, H_), F32),
            jax.ShapeDtypeStruct((N1_, 1), F32),
            jax.ShapeDtypeStruct((N1_, 1), F32),
        ],
    )(acc1, b1.reshape(1, H_), W2, a2s.reshape(H_, 1), a2d.reshape(H_, 1))

    acc2 = _edge_l2(hs2, as2.reshape(N1_), ad2.reshape(N1_)[:N2_], src2, dst2)

    out = pl.pallas_call(
        _final_dense_body,
        out_shape=jax.ShapeDtypeStruct((N2_, C_), F32),
    )(acc2, b2.reshape(1, H_), Wl, bl.reshape(1, C_))
    return out


# speedup vs baseline: 1.0052x; 1.0052x over previous
import functools

import jax
import jax.numpy as jnp
from jax import lax
from jax.experimental import pallas as pl
from jax.experimental.pallas import tpu as pltpu
from jax.experimental.pallas import tpu_sc as plsc

F32 = jnp.float32
N0_, N1_, N2_ = 10000, 4000, 1024
E1_, E2_ = 320000, 128000
F_IN_, H_, C_ = 128, 64, 10

NC, NS = 2, 16
NW = NC * NS
RW = 80


def _l1_dense_body(x_ref, w_ref, avs_ref, avd_ref, hs_ref, as_ref, ad_ref):
    hs = jnp.dot(x_ref[...], w_ref[...], preferred_element_type=F32)
    hs_ref[...] = hs
    as_ref[...] = jnp.dot(hs, avs_ref[...], preferred_element_type=F32)
    ad_ref[...] = jnp.dot(hs, avd_ref[...], preferred_element_type=F32)


def _mid_dense_body(acc_ref, b1_ref, w2_ref, avs_ref, avd_ref,
                    hs2_ref, as2_ref, ad2_ref):
    a0 = acc_ref[0, :N1_]
    a1 = acc_ref[1, :N1_]
    num = a0[:, :H_] + a1[:, :H_]
    den = a0[:, H_:H_ + 1] + a1[:, H_:H_ + 1]
    h = jnp.maximum(num / (den + 1e-16) + b1_ref[...], 0.0)
    hs2 = jnp.dot(h, w2_ref[...], preferred_element_type=F32)
    hs2_ref[...] = hs2
    as2_ref[...] = jnp.dot(hs2, avs_ref[...], preferred_element_type=F32)
    ad2_ref[...] = jnp.dot(hs2, avd_ref[...], preferred_element_type=F32)


def _final_dense_body(acc_ref, b2_ref, wl_ref, bl_ref, out_ref):
    a0 = acc_ref[0, :N2_]
    a1 = acc_ref[1, :N2_]
    num = a0[:, :H_] + a1[:, :H_]
    den = a0[:, H_:H_ + 1] + a1[:, H_:H_ + 1]
    h2 = num / (den + 1e-16) + b2_ref[...]
    logits = jnp.dot(h2, wl_ref[...], preferred_element_type=F32) + bl_ref[...]
    m = jnp.max(logits, axis=1, keepdims=True)
    lse = m + jnp.log(jnp.sum(jnp.exp(logits - m), axis=1, keepdims=True))
    out_ref[...] = logits - lse


def _make_edge_kernel(n_src, n_dst, n_edges, chunk, nbuf):
    ew = n_edges // NW
    n_chunks = ew // chunk
    assert ew % chunk == 0 and chunk % 8 == 0
    nq = n_chunks // nbuf
    rem = n_chunks % nbuf
    n_dst_pad = (n_dst + 127) // 128 * 128
    rpt = n_dst_pad // NS
    ZR = 64
    mesh = plsc.VectorSubcoreMesh(core_axis_name="c", subcore_axis_name="s")

    @functools.partial(
        pl.kernel,
        out_type=jax.ShapeDtypeStruct((NC, n_dst_pad, RW), F32),
        mesh=mesh,
        compiler_params=pltpu.CompilerParams(
            needs_layout_passes=False, use_tc_tiling_on_sc=False),
        scratch_types=[
            pltpu.VMEM((n_src,), F32),
            pltpu.VMEM((n_dst,), F32),
            [pltpu.VMEM((chunk,), jnp.int32)] * nbuf,
            [pltpu.VMEM((chunk,), jnp.int32)] * nbuf,
            [pltpu.VMEM((chunk, H_), F32)] * nbuf,
            [pltpu.VMEM((chunk, RW), F32)] * nbuf,
            pltpu.VMEM((ZR, RW), F32),
            pltpu.VMEM_SHARED((n_dst_pad, RW), F32),
            [pltpu.SemaphoreType.DMA] * nbuf,
            [pltpu.SemaphoreType.DMA] * nbuf,
        ],
    )
    def edge_kernel(hs_hbm, asrc_hbm, adst_hbm, src_hbm, dst_hbm, out_hbm,
                    asrc_v, adst_v, src_v, dst_v, rows_v, srows_v,
                    zero_v, acc_sh, gsem, ssem):
        c = lax.axis_index("c")
        s = lax.axis_index("s")
        wid = s * NC + c

        pltpu.sync_copy(asrc_hbm, asrc_v)
        pltpu.sync_copy(adst_hbm, adst_v)
        zv = jnp.zeros((16,), F32)

        def zrow(r, _):
            for col in range(RW // 16):
                zero_v[r, pl.ds(col * 16, 16)] = zv
            return 0
        lax.fori_loop(0, ZR, zrow, 0)

        def zpad(r, _):
            for b in range(nbuf):
                srows_v[b][r, pl.ds(H_, 16)] = zv
            return 0
        lax.fori_loop(0, chunk, zpad, 0)

        def zacc(i, _):
            pltpu.sync_copy(zero_v, acc_sh.at[pl.ds(s * rpt + i * ZR, ZR)])
            return 0
        lax.fori_loop(0, rpt // ZR, zacc, 0)
        plsc.subcore_barrier()

        lane = lax.iota(jnp.int32, 16)
        col64 = jnp.full((16,), H_, jnp.int32)
        e0 = wid * ew

        def load_idx(g, b):
            base = e0 + g * chunk
            pltpu.sync_copy(src_hbm.at[pl.ds(base, chunk)], src_v[b])
            pltpu.sync_copy(dst_hbm.at[pl.ds(base, chunk)], dst_v[b])

        def scale(b):
            def group(b16):
                s16 = src_v[b][pl.ds(b16, 16)]
                d16 = dst_v[b][pl.ds(b16, 16)]
                e = plsc.load_gather(asrc_v, [s16]) + plsc.load_gather(adst_v, [d16])
                e = jnp.where(e >= 0.0, e, e * 0.2)
                ex16 = jnp.exp(e)
                nc = H_ // 16
                for r0 in range(0, 16, 4):
                    vals = [rows_v[b][b16 + r0 + q, pl.ds(c * 16, 16)]
                            for q in range(4) for c in range(nc)]
                    prods = [vals[q * nc + c] * ex16[r0 + q]
                             for q in range(4) for c in range(nc)]
                    for q in range(4):
                        for c in range(nc):
                            srows_v[b][b16 + r0 + q, pl.ds(c * 16, 16)] = (
                                prods[q * nc + c])
                plsc.store_scatter(srows_v[b], [b16 + lane, col64], ex16)

            @plsc.parallel_loop(0, chunk // 16, unroll=4)
            def scale_body(j):
                group(j * 16)
            if chunk % 16:
                group(chunk - 16)

        def rotation(g0, nb):
            gathers = []
            for b in range(nb):
                load_idx(g0 + b, b)
                gathers.append(
                    pltpu.async_copy(hs_hbm.at[src_v[b]], rows_v[b], gsem[b]))
            scatters = []
            for b in range(nb):
                gathers[b].wait()
                scale(b)
                scatters.append(
                    pltpu.async_copy(srows_v[b], acc_sh.at[dst_v[b]], ssem[b],
                                     add=True))
            for b in range(nb):
                scatters[b].wait()

        def rot_body(t, _):
            rotation(nbuf * t, nbuf)
            return 0
        lax.fori_loop(0, nq, rot_body, 0)
        if rem:
            rotation(nq * nbuf, rem)

        plsc.subcore_barrier()

        def wb(i, _):
            pltpu.sync_copy(acc_sh.at[pl.ds(s * rpt + i * ZR, ZR)],
                            out_hbm.at[c, pl.ds(s * rpt + i * ZR, ZR)])
            return 0
        lax.fori_loop(0, rpt // ZR, wb, 0)

    return edge_kernel


_edge_l1 = _make_edge_kernel(N0_, N1_, E1_, 200, 3)
_edge_l2 = _make_edge_kernel(N1_, N2_, E2_, 200, 3)


def kernel(x, src1, dst1, src2, dst2, W1, a1s, a1d, b1, W2, a2s, a2d, b2, Wl, bl):
    hs1, as1, ad1 = pl.pallas_call(
        _l1_dense_body,
        out_shape=[
            jax.ShapeDtypeStruct((N0_, H_), F32),
            jax.ShapeDtypeStruct((N0_, 1), F32),
            jax.ShapeDtypeStruct((N0_, 1), F32),
        ],
    )(x, W1, a1s.reshape(H_, 1), a1d.reshape(H_, 1))

    acc1 = _edge_l1(hs1, as1.reshape(N0_), ad1.reshape(N0_)[:N1_], src1, dst1)

    hs2, as2, ad2 = pl.pallas_call(
        _mid_dense_body,
        out_shape=[
            jax.ShapeDtypeStruct((N1_, H_), F32),
            jax.ShapeDtypeStruct((N1_, 1), F32),
            jax.ShapeDtypeStruct((N1_, 1), F32),
        ],
    )(acc1, b1.reshape(1, H_), W2, a2s.reshape(H_, 1), a2d.reshape(H_, 1))

    acc2 = _edge_l2(hs2, as2.reshape(N1_), ad2.reshape(N1_)[:N2_], src2, dst2)

    out = pl.pallas_call(
        _final_dense_body,
        out_shape=jax.ShapeDtypeStruct((N2_, C_), F32),
    )(acc2, b2.reshape(1, H_), Wl, bl.reshape(1, C_))
    return out
